# baseline (speedup 1.0000x reference)
import functools

import jax
import jax.numpy as jnp
from jax import lax
from jax.experimental import pallas as pl
from jax.experimental.pallas import tpu as pltpu
from jax.experimental.pallas import tpu_sc as plsc

NN = 10000
NE = 320000
ND = 128
HD = 128
NC = 2
NS = 16
NW = NC * NS
EPW = 10240
NEP = NW * EPW
C = 32
NCH = EPW // C
IPR = 128 // C
NQ = NN + 16
OWN = 624
TAIL = NN - NS * OWN
ZTAIL = NQ - NS * OWN
VPR = HD // 16

import numpy as np
PERM = np.array(
    [g * 32 + off
     for g in range(HD // 32)
     for i in range(16)
     for off in (i, 16 + i)],
    dtype=np.int32,
)


def _pq_body(x_ref, ws_ref, wd_ref, p_ref, q_ref):
    x = x_ref[...]
    p_ref[...] = jnp.dot(x, ws_ref[...], preferred_element_type=jnp.float32)
    q_ref[...] = jnp.dot(x, wd_ref[...], preferred_element_type=jnp.float32)


_pq = pl.pallas_call(
    _pq_body,
    grid=(10,),
    in_specs=[
        pl.BlockSpec((1000, ND), lambda i: (i, 0)),
        pl.BlockSpec((ND, HD), lambda i: (0, 0)),
        pl.BlockSpec((ND, HD), lambda i: (0, 0)),
    ],
    out_specs=[
        pl.BlockSpec((1000, HD), lambda i: (i, 0)),
        pl.BlockSpec((1000, HD), lambda i: (i, 0)),
    ],
    out_shape=[
        jax.ShapeDtypeStruct((NN, HD), jnp.float32),
        jax.ShapeDtypeStruct((NN, HD), jnp.float32),
    ],
)


def _ep_body(e_ref, we_ref, b1_ref, o_ref):
    o_ref[...] = (
        jnp.dot(e_ref[...], we_ref[...], preferred_element_type=jnp.float32)
        + b1_ref[0:1, :]
    )


_ep = pl.pallas_call(
    _ep_body,
    grid=(80,),
    in_specs=[
        pl.BlockSpec((4096, 16), lambda i: (i, 0)),
        pl.BlockSpec((16, HD), lambda i: (0, 0)),
        pl.BlockSpec((8, HD), lambda i: (0, 0)),
    ],
    out_specs=pl.BlockSpec((4096, HD), lambda i: (i, 0)),
    out_shape=jax.ShapeDtypeStruct((NEP, HD), jnp.float32),
)


def _pack_body(s_ref, d_ref, o_ref):
    o_ref[...] = (s_ref[...] << 16) | d_ref[...]


_pack = pl.pallas_call(
    _pack_body,
    grid=(4,),
    in_specs=[
        pl.BlockSpec((8, EPW), lambda i: (i, 0)),
        pl.BlockSpec((8, EPW), lambda i: (i, 0)),
    ],
    out_specs=pl.BlockSpec((8, EPW), lambda i: (i, 0)),
    out_shape=jax.ShapeDtypeStruct((NW, EPW), jnp.int32),
)


def _out_body(x_ref, a0_ref, a1_ref, wx_ref, wa_ref, b2_ref, o_ref):
    acc = jnp.dot(x_ref[...], wx_ref[...], preferred_element_type=jnp.float32)
    acc = acc + jnp.dot(
        a0_ref[...] + a1_ref[...], wa_ref[...],
        preferred_element_type=jnp.float32,
    )
    o_ref[...] = jnp.maximum(acc + b2_ref[0:1, :], 0.0)


_outk = pl.pallas_call(
    _out_body,
    grid=(10,),
    in_specs=[
        pl.BlockSpec((1000, ND), lambda i: (i, 0)),
        pl.BlockSpec((1000, HD), lambda i: (i, 0)),
        pl.BlockSpec((1000, HD), lambda i: (i, 0)),
        pl.BlockSpec((ND, ND), lambda i: (0, 0)),
        pl.BlockSpec((HD, ND), lambda i: (0, 0)),
        pl.BlockSpec((8, ND), lambda i: (0, 0)),
    ],
    out_specs=pl.BlockSpec((1000, ND), lambda i: (i, 0)),
    out_shape=jax.ShapeDtypeStruct((NN, ND), jnp.float32),
)


def _sc_body(p_hbm, q_hbm, e_hbm, pidx_hbm, out_hbm,
             pidx, sidx, didx, bufp, bufq, bufe, bufm, agg_sh,
             sem_p0, sem_q0, sem_e0, sem_p1, sem_q1, sem_e1, sem_w):
    cid = lax.axis_index("c")
    sid = lax.axis_index("s")
    wid = sid * NC + cid


    zero = jnp.zeros((16,), jnp.float32)

    def zrow(r, carry):
        for j in range(VPR):
            bufm[r, pl.ds(j * 16, 16)] = zero
        return carry

    lax.fori_loop(0, C, zrow, 0)

    def zcopy(k, carry):
        pltpu.sync_copy(bufm, agg_sh.at[pl.ds(sid * OWN + k * C, C)])
        return carry

    lax.fori_loop(0, OWN // C, zcopy, 0)
    pltpu.sync_copy(
        bufm.at[pl.ds(0, OWN % C)],
        agg_sh.at[pl.ds(sid * OWN + (OWN // C) * C, OWN % C)],
    )

    @pl.when(sid == NS - 1)
    def _():
        pltpu.sync_copy(
            bufm.at[pl.ds(0, ZTAIL)], agg_sh.at[pl.ds(NS * OWN, ZTAIL)]
        )

    pltpu.sync_copy(pidx_hbm.at[wid], pidx)

    plsc.subcore_barrier()

    def issue(c, st):
        pltpu.async_copy(p_hbm.at[sidx.at[st]], bufp.at[st],
                         (sem_p0, sem_p1)[st])
        pltpu.async_copy(q_hbm.at[didx.at[st]], bufq.at[st],
                         (sem_q0, sem_q1)[st])
        pltpu.async_copy(e_hbm.at[pl.ds(wid * EPW + c * C, C)],
                         bufe.at[st], (sem_e0, sem_e1)[st])

    def wait_loads(c, st):
        pltpu.make_async_copy(p_hbm.at[sidx.at[st]], bufp.at[st],
                              (sem_p0, sem_p1)[st]).wait()
        pltpu.make_async_copy(q_hbm.at[didx.at[st]], bufq.at[st],
                              (sem_q0, sem_q1)[st]).wait()
        pltpu.make_async_copy(e_hbm.at[pl.ds(wid * EPW + c * C, C)],
                              bufe.at[st], (sem_e0, sem_e1)[st]).wait()

    def unpack_idx(c, st):
        r = c // IPR
        h = c % IPR
        for j in range(C // 16):
            w = pidx[r, pl.ds(h * C + j * 16, 16)]
            sidx[st, pl.ds(j * 16, 16)] = w >> 16
            didx[st, pl.ds(j * 16, 16)] = w & 0xFFFF

    def process(c, carry):
        st = lax.rem(c, 2)
        @pl.when(c > 0)
        def _():
            pltpu.make_async_copy(bufm, agg_sh.at[didx.at[st]], sem_w).wait()

        @pl.when(jnp.logical_and(c + 1 < NCH, st == 0))
        def _():
            unpack_idx(c + 1, 1)
            issue(c + 1, 1)

        @pl.when(jnp.logical_and(c + 1 < NCH, st == 1))
        def _():
            unpack_idx(c + 1, 0)
            issue(c + 1, 0)

        @pl.when(st == 0)
        def _():
            wait_loads(c, 0)

        @pl.when(st == 1)
        def _():
            wait_loads(c, 1)

        def row(rr, rc):
            for j in range(VPR):
                s = pl.ds(j * 16, 16)
                v = bufp[st, rr, s] + bufq[st, rr, s] + bufe[st, rr, s]
                bufm[rr, s] = jnp.maximum(v, 0.0)
            return rc

        lax.fori_loop(0, C, row, 0)

        pltpu.async_copy(bufm, agg_sh.at[didx.at[st]], sem_w, add=True)
        return carry

    unpack_idx(0, 0)

    @pl.when(cid >= 0)
    def _():
        issue(0, 0)

    lax.fori_loop(0, NCH, process, 0)

    pltpu.make_async_copy(bufm, agg_sh.at[didx.at[1]], sem_w).wait()

    plsc.subcore_barrier()

    pltpu.sync_copy(
        agg_sh.at[pl.ds(sid * OWN, OWN)],
        out_hbm.at[pl.ds(cid * NN + sid * OWN, OWN)],
    )

    @pl.when(sid == NS - 1)
    def _():
        pltpu.sync_copy(
            agg_sh.at[pl.ds(NS * OWN, TAIL)],
            out_hbm.at[pl.ds(cid * NN + NS * OWN, TAIL)],
        )


_sc_agg = functools.partial(
    pl.kernel,
    out_type=jax.ShapeDtypeStruct((NC * NN, HD), jnp.float32),
    mesh=plsc.VectorSubcoreMesh(core_axis_name="c", subcore_axis_name="s"),
    scratch_types=[
        pltpu.VMEM((EPW // 128, 128), jnp.int32),
        pltpu.VMEM((2, C), jnp.int32),
        pltpu.VMEM((2, C), jnp.int32),
        pltpu.VMEM((2, C, HD), jnp.float32),
        pltpu.VMEM((2, C, HD), jnp.float32),
        pltpu.VMEM((2, C, HD), jnp.float32),
        pltpu.VMEM((C, HD), jnp.float32),
        pltpu.VMEM_SHARED((NQ, HD), jnp.float32),
        pltpu.SemaphoreType.DMA,
        pltpu.SemaphoreType.DMA,
        pltpu.SemaphoreType.DMA,
        pltpu.SemaphoreType.DMA,
        pltpu.SemaphoreType.DMA,
        pltpu.SemaphoreType.DMA,
        pltpu.SemaphoreType.DMA,
    ],
)(_sc_body)


def kernel(node_features, edge_features, edge_index, W1, b1, W2, b2):
    w1s = W1[:ND]
    w1d = W1[ND:2 * ND]
    w1e = W1[2 * ND:]
    w2x = W2[:ND]
    w2a = W2[ND:]
    b1t = jnp.broadcast_to(b1[None, :], (8, HD))
    b2t = jnp.broadcast_to(b2[None, :], (8, ND))

    p, q = _pq(node_features, w1s, w1d)
    q = jnp.pad(q, ((0, NQ - NN), (0, 0)))

    pad = EPW - NE // NW
    e_pad = jnp.pad(
        edge_features.reshape(NW, NE // NW, 16), ((0, 0), (0, pad), (0, 0))
    ).reshape(NEP, 16)
    ep = _ep(e_pad, w1e, b1t)

    src_p = jnp.pad(edge_index[0].reshape(NW, NE // NW), ((0, 0), (0, pad)))
    dst_p = jnp.pad(edge_index[1].reshape(NW, NE // NW), ((0, 0), (0, pad)),
                    constant_values=NN)
    pidx = _pack(src_p, dst_p).reshape(NW, EPW // 128, 128)

    aggs = _sc_agg(p, q, ep, pidx)
    return _outk(node_features, aggs[:NN], aggs[NN:], w2x, w2a, b2t)


# speedup vs baseline: 1.4030x; 1.4030x over previous
import functools

import jax
import jax.numpy as jnp
from jax import lax
from jax.experimental import pallas as pl
from jax.experimental.pallas import tpu as pltpu
from jax.experimental.pallas import tpu_sc as plsc

NN = 10000
NE = 320000
ND = 128
HD = 128
NC = 2
NS = 16
NW = NC * NS
EPW = 10240
NEP = NW * EPW
C = 32
NCH = EPW // C
IPR = 128 // C
NQ = NN + 16
OWN = 624
TAIL = NN - NS * OWN
ZTAIL = NQ - NS * OWN
VPR = HD // 16

import numpy as np
PERM = np.array(
    [g * 32 + off
     for g in range(HD // 32)
     for i in range(16)
     for off in (i, 16 + i)],
    dtype=np.int32,
)


def _pq_body(x_ref, ws_ref, wd_ref, p_ref, q_ref):
    x = x_ref[...]
    p_ref[...] = jnp.dot(x, ws_ref[...], preferred_element_type=jnp.float32)
    q_ref[...] = jnp.dot(x, wd_ref[...], preferred_element_type=jnp.float32)


_pq = pl.pallas_call(
    _pq_body,
    grid=(10,),
    in_specs=[
        pl.BlockSpec((1000, ND), lambda i: (i, 0)),
        pl.BlockSpec((ND, HD), lambda i: (0, 0)),
        pl.BlockSpec((ND, HD), lambda i: (0, 0)),
    ],
    out_specs=[
        pl.BlockSpec((1000, HD), lambda i: (i, 0)),
        pl.BlockSpec((1000, HD), lambda i: (i, 0)),
    ],
    out_shape=[
        jax.ShapeDtypeStruct((NN, HD), jnp.float32),
        jax.ShapeDtypeStruct((NN, HD), jnp.float32),
    ],
)


def _ep_body(e_ref, we_ref, b1_ref, o_ref):
    o_ref[...] = (
        jnp.dot(e_ref[...], we_ref[...], preferred_element_type=jnp.float32)
        + b1_ref[0:1, :]
    )


_ep = pl.pallas_call(
    _ep_body,
    grid=(80,),
    in_specs=[
        pl.BlockSpec((4096, 16), lambda i: (i, 0)),
        pl.BlockSpec((16, HD), lambda i: (0, 0)),
        pl.BlockSpec((8, HD), lambda i: (0, 0)),
    ],
    out_specs=pl.BlockSpec((4096, HD), lambda i: (i, 0)),
    out_shape=jax.ShapeDtypeStruct((NEP, HD), jnp.float32),
)


def _pack_body(s_ref, d_ref, o_ref):
    o_ref[...] = (s_ref[...] << 16) | d_ref[...]


_pack = pl.pallas_call(
    _pack_body,
    grid=(4,),
    in_specs=[
        pl.BlockSpec((8, EPW), lambda i: (i, 0)),
        pl.BlockSpec((8, EPW), lambda i: (i, 0)),
    ],
    out_specs=pl.BlockSpec((8, EPW), lambda i: (i, 0)),
    out_shape=jax.ShapeDtypeStruct((NW, EPW), jnp.int32),
)


def _out_body(x_ref, a0_ref, a1_ref, wx_ref, wa_ref, b2_ref, o_ref):
    acc = jnp.dot(x_ref[...], wx_ref[...], preferred_element_type=jnp.float32)
    acc = acc + jnp.dot(
        a0_ref[...] + a1_ref[...], wa_ref[...],
        preferred_element_type=jnp.float32,
    )
    o_ref[...] = jnp.maximum(acc + b2_ref[0:1, :], 0.0)


_outk = pl.pallas_call(
    _out_body,
    grid=(10,),
    in_specs=[
        pl.BlockSpec((1000, ND), lambda i: (i, 0)),
        pl.BlockSpec((1000, HD), lambda i: (i, 0)),
        pl.BlockSpec((1000, HD), lambda i: (i, 0)),
        pl.BlockSpec((ND, ND), lambda i: (0, 0)),
        pl.BlockSpec((HD, ND), lambda i: (0, 0)),
        pl.BlockSpec((8, ND), lambda i: (0, 0)),
    ],
    out_specs=pl.BlockSpec((1000, ND), lambda i: (i, 0)),
    out_shape=jax.ShapeDtypeStruct((NN, ND), jnp.float32),
)


def _sc_body(t_hbm, e_hbm, pidx_hbm, out_hbm,
             pidx, sidx, didx, bpq, bufe, bufm, agg_sh,
             sem_l0, sem_l1, sem_w):
    cid = lax.axis_index("c")
    sid = lax.axis_index("s")
    wid = sid * NC + cid

    zero = jnp.zeros((16,), jnp.float32)

    def zrow(r, carry):
        for j in range(VPR):
            bufm[r, pl.ds(j * 16, 16)] = zero
        return carry

    lax.fori_loop(0, C, zrow, 0)

    def zcopy(k, carry):
        pltpu.sync_copy(bufm, agg_sh.at[pl.ds(sid * OWN + k * C, C)])
        return carry

    lax.fori_loop(0, OWN // C, zcopy, 0)
    pltpu.sync_copy(
        bufm.at[pl.ds(0, OWN % C)],
        agg_sh.at[pl.ds(sid * OWN + (OWN // C) * C, OWN % C)],
    )

    @pl.when(sid == NS - 1)
    def _():
        pltpu.sync_copy(
            bufm.at[pl.ds(0, ZTAIL)], agg_sh.at[pl.ds(NS * OWN, ZTAIL)]
        )

    pltpu.sync_copy(pidx_hbm.at[wid], pidx.at[pl.ds(0, EPW // 128)])

    plsc.subcore_barrier()

    def unpack_idx(c, st):
        r = c // IPR
        h = c % IPR
        for j in range(C // 16):
            w = pidx[r, pl.ds(h * C + j * 16, 16)]
            d = w & 0xFFFF
            sidx[st, pl.ds(j * 16, 16)] = w >> 16
            sidx[st, pl.ds(C + j * 16, 16)] = d + NN
            didx[st, pl.ds(j * 16, 16)] = d

    def issue(c, st):
        sem = (sem_l0, sem_l1)[st]
        pltpu.async_copy(t_hbm.at[sidx.at[st]], bpq.at[st], sem)
        pltpu.async_copy(e_hbm.at[pl.ds(wid * EPW + c * C, C)],
                         bufe.at[st], sem)

    def wait_loads(c, st):
        sem = (sem_l0, sem_l1)[st]
        pltpu.make_async_copy(t_hbm.at[sidx.at[st]], bpq.at[st], sem).wait()
        pltpu.make_async_copy(e_hbm.at[pl.ds(wid * EPW + c * C, C)],
                              bufe.at[st], sem).wait()

    def process(c, carry):
        st = lax.rem(c, 2)
        @pl.when(c > 0)
        def _():
            pltpu.make_async_copy(bufm, agg_sh.at[didx.at[st]], sem_w).wait()

        unpack_idx(c + 1, 1 - st)

        @pl.when(jnp.logical_and(c + 1 < NCH, st == 0))
        def _():
            issue(c + 1, 1)

        @pl.when(jnp.logical_and(c + 1 < NCH, st == 1))
        def _():
            issue(c + 1, 0)

        @pl.when(st == 0)
        def _():
            wait_loads(c, 0)

        @pl.when(st == 1)
        def _():
            wait_loads(c, 1)

        @plsc.parallel_loop(0, C, unroll=4)
        def _(rr):
            for j in range(VPR):
                s = pl.ds(j * 16, 16)
                v = bpq[st, rr, s] + bpq[st, C + rr, s] + bufe[st, rr, s]
                bufm[rr, s] = jnp.maximum(v, 0.0)

        pltpu.async_copy(bufm, agg_sh.at[didx.at[st]], sem_w, add=True)
        return carry

    unpack_idx(0, 0)
    issue(0, 0)

    lax.fori_loop(0, NCH, process, 0)

    pltpu.make_async_copy(bufm, agg_sh.at[didx.at[1]], sem_w).wait()

    plsc.subcore_barrier()

    pltpu.sync_copy(
        agg_sh.at[pl.ds(sid * OWN, OWN)],
        out_hbm.at[pl.ds(cid * NN + sid * OWN, OWN)],
    )

    @pl.when(sid == NS - 1)
    def _():
        pltpu.sync_copy(
            agg_sh.at[pl.ds(NS * OWN, TAIL)],
            out_hbm.at[pl.ds(cid * NN + NS * OWN, TAIL)],
        )


_sc_agg = functools.partial(
    pl.kernel,
    out_type=jax.ShapeDtypeStruct((NC * NN, HD), jnp.float32),
    mesh=plsc.VectorSubcoreMesh(core_axis_name="c", subcore_axis_name="s"),
    scratch_types=[
        pltpu.VMEM((EPW // 128 + 8, 128), jnp.int32),
        pltpu.VMEM((2, 2 * C), jnp.int32),
        pltpu.VMEM((2, C), jnp.int32),
        pltpu.VMEM((2, 2 * C, HD), jnp.float32),
        pltpu.VMEM((2, C, HD), jnp.float32),
        pltpu.VMEM((C, HD), jnp.float32),
        pltpu.VMEM_SHARED((NQ, HD), jnp.float32),
        pltpu.SemaphoreType.DMA,
        pltpu.SemaphoreType.DMA,
        pltpu.SemaphoreType.DMA,
    ],
)(_sc_body)


def kernel(node_features, edge_features, edge_index, W1, b1, W2, b2):
    w1s = W1[:ND]
    w1d = W1[ND:2 * ND]
    w1e = W1[2 * ND:]
    w2x = W2[:ND]
    w2a = W2[ND:]
    b1t = jnp.broadcast_to(b1[None, :], (8, HD))
    b2t = jnp.broadcast_to(b2[None, :], (8, ND))

    p, q = _pq(node_features, w1s, w1d)
    t = jnp.concatenate([p, jnp.pad(q, ((0, NQ - NN), (0, 0)))])

    pad = EPW - NE // NW
    e_pad = jnp.pad(
        edge_features.reshape(NW, NE // NW, 16), ((0, 0), (0, pad), (0, 0))
    ).reshape(NEP, 16)
    ep = _ep(e_pad, w1e, b1t)

    src_p = jnp.pad(edge_index[0].reshape(NW, NE // NW), ((0, 0), (0, pad)))
    dst_p = jnp.pad(edge_index[1].reshape(NW, NE // NW), ((0, 0), (0, pad)),
                    constant_values=NN)
    pidx = _pack(src_p, dst_p).reshape(NW, EPW // 128, 128)

    aggs = _sc_agg(t, ep, pidx)
    return _outk(node_features, aggs[:NN], aggs[NN:], w2x, w2a, b2t)
